# baseline (speedup 1.0000x reference)
import functools

import jax
import jax.numpy as jnp
from jax import lax
from jax.experimental import pallas as pl
from jax.experimental.pallas import tpu as pltpu
from jax.experimental.pallas import tpu_sc as plsc

_B, _L, _D = 4096, 50, 32
_NC, _NS = 2, 16
_NW = _NC * _NS
_BPW = _B // _NW
_IPW = _BPW * _L
_QOFF = (0, 13, 26, 38)
_QLEN = (13, 13, 12, 12)
_NSL = 8
_SCALE = 1.0 / _L

_mesh = plsc.VectorSubcoreMesh(
    core_axis_name="c", subcore_axis_name="s", num_cores=_NC, num_subcores=_NS
)


@functools.partial(
    pl.kernel,
    out_type=jax.ShapeDtypeStruct((_B * _D,), jnp.float32),
    mesh=_mesh,
    scratch_types=[
        pltpu.VMEM((_IPW + 16,), jnp.int32),
        pltpu.VMEM((_BPW * _D,), jnp.float32),
        pltpu.VMEM((_NSL, 16), jnp.int32),
        pltpu.VMEM((_NSL, 16, 128), jnp.float32),
        pltpu.SemaphoreType.DMA((_NSL,)),
    ],
    compiler_params=pltpu.CompilerParams(needs_layout_passes=False),
)
def _lookup_mean(idx_hbm, table_hbm, out_hbm, idx_v, out_v, lists_v, ring_v, sems):
    wid = lax.axis_index("s") * _NC + lax.axis_index("c")
    pltpu.sync_copy(idx_hbm.at[pl.ds(wid * _IPW, _IPW)], idx_v.at[pl.ds(0, _IPW)])
    iota = lax.iota(jnp.int32, 16)

    def _chunk(b, c):
        off = b * _L + _QOFF[c]
        return plsc.load_gather(idx_v, [jnp.full((16,), off, jnp.int32) + iota])

    def _issue(b, c, s):
        v = _chunk(b, c)
        lists_v[s, :] = v >> 2
        pltpu.async_copy(
            table_hbm.at[lists_v.at[s, pl.ds(0, _QLEN[c])]],
            ring_v.at[s, pl.ds(0, _QLEN[c])],
            sems.at[s],
        )

    def _consume(b, c, s, a0, a1):
        pltpu.make_async_copy(
            table_hbm.at[lists_v.at[s, pl.ds(0, _QLEN[c])]],
            ring_v.at[s, pl.ds(0, _QLEN[c])],
            sems.at[s],
        ).wait()
        v = _chunk(b, c)
        sub = v & 3
        for i in range(_QLEN[c]):
            si = sub[i] * 32
            a0 = a0 + ring_v[s, i, pl.ds(si, 16)]
            a1 = a1 + ring_v[s, i, pl.ds(si + 16, 16)]
        return a0, a1

    for p in range(2):
        for c in range(4):
            _issue(p, c, 4 * p + c)

    @pl.loop(0, _BPW - 2, step=2)
    def _main(b):
        for p in range(2):
            a0 = jnp.zeros((16,), jnp.float32)
            a1 = jnp.zeros((16,), jnp.float32)
            for c in range(4):
                s = 4 * p + c
                a0, a1 = _consume(b + p, c, s, a0, a1)
                _issue(b + p + 2, c, s)
            out_v[pl.ds((b + p) * _D, 16)] = a0 * _SCALE
            out_v[pl.ds((b + p) * _D + 16, 16)] = a1 * _SCALE

    for p in range(2):
        b = _BPW - 2 + p
        a0 = jnp.zeros((16,), jnp.float32)
        a1 = jnp.zeros((16,), jnp.float32)
        for c in range(4):
            a0, a1 = _consume(b, c, 4 * p + c, a0, a1)
        out_v[pl.ds(b * _D, 16)] = a0 * _SCALE
        out_v[pl.ds(b * _D + 16, 16)] = a1 * _SCALE

    pltpu.sync_copy(out_v, out_hbm.at[pl.ds(wid * _BPW * _D, _BPW * _D)])


def kernel(idx, embedding):
    idx1d = idx.reshape(-1)
    table4 = embedding.reshape(250000, 128)
    out = _lookup_mean(idx1d, table4)
    return out.reshape(_B, 1, _D)
